# Initial kernel scaffold; baseline (speedup 1.0000x reference)
import jax
import jax.numpy as jnp
from jax.experimental import pallas as pl


def kernel(z, edge_distance, edge_index, a_factor, Z_power, screen_coefs, screen_exps, covalent_radii):
    raise NotImplementedError("write your pallas kernel here")


# speedup vs baseline: 288.6823x; 288.6823x over previous
import jax
import jax.numpy as jnp
from jax import lax
from jax.experimental import pallas as pl
from jax.experimental.pallas import tpu as pltpu
from jax.experimental.pallas import tpu_sc as plsc

N_NODES = 100000
N_EDGES = 6400000
NC, NS, LANES = 2, 16, 16
NW = NC * NS
ROW = 128
CH_ROWS = 16
CH = CH_ROWS * ROW
CHUNKS = 98
TILE_E = CH * CHUNKS
EP = TILE_E * NW
ROWS = EP // ROW
TILE_ROWS = TILE_E // ROW
SEG = 6256
ACC_N = SEG * NS
TAB = 128


def _zbl_body(z_hbm, send_hbm, recv_hbm, dist_hbm, zpow_hbm, rad_hbm, par_hbm,
              out_hbm,
              z_v, zpow_v, rad_v, par_v, s_v, r_v, d_v, o_v, zero_v, acc_sh):
    cid = lax.axis_index("c")
    sid = lax.axis_index("s")
    wid = sid * NC + cid

    pltpu.sync_copy(z_hbm, z_v)
    pltpu.sync_copy(zpow_hbm, zpow_v)
    pltpu.sync_copy(rad_hbm, rad_v)
    pltpu.sync_copy(par_hbm, par_v)

    def _zero(i, c):
        zero_v[pl.ds(i * LANES, LANES)] = jnp.zeros((LANES,), jnp.float32)
        return c
    lax.fori_loop(0, SEG // LANES, _zero, 0)
    pltpu.sync_copy(zero_v, acc_sh.at[pl.ds(sid * SEG, SEG)])
    plsc.subcore_barrier()

    inv_ab = par_v[0]
    c0, c1, c2, c3 = par_v[1], par_v[2], par_v[3], par_v[4]
    nd0, nd1, nd2, nd3 = par_v[5], par_v[6], par_v[7], par_v[8]

    row0 = wid * TILE_ROWS

    def _chunk(k, carry):
        rb = row0 + k * CH_ROWS
        pltpu.sync_copy(send_hbm.at[pl.ds(rb, CH_ROWS)], s_v)
        pltpu.sync_copy(recv_hbm.at[pl.ds(rb, CH_ROWS)], r_v)
        pltpu.sync_copy(dist_hbm.at[pl.ds(rb, CH_ROWS)], d_v)

        def _vec(i, c):
            row = i // (ROW // LANES)
            col = (i % (ROW // LANES)) * LANES
            s = s_v[row, pl.ds(col, LANES)]
            r = r_v[row, pl.ds(col, LANES)]
            dd = d_v[row, pl.ds(col, LANES)]
            zu = plsc.load_gather(z_v, [s])
            zv = plsc.load_gather(z_v, [r])
            zfu = zu.astype(jnp.float32)
            zfv = zv.astype(jnp.float32)
            pu = plsc.load_gather(zpow_v, [zu])
            pv = plsc.load_gather(zpow_v, [zv])
            ru = plsc.load_gather(rad_v, [zu])
            rv = plsc.load_gather(rad_v, [zv])
            x = dd * ((pu + pv) * inv_ab)
            phi = (c0 * jnp.exp(nd0 * x) + c1 * jnp.exp(nd1 * x)
                   + c2 * jnp.exp(nd2 * x) + c3 * jnp.exp(nd3 * x))
            y = dd / (ru + rv)
            y2 = y * y
            y4 = y2 * y2
            y6 = y4 * y2
            env = 1.0 - 28.0 * y6 + 48.0 * y6 * y - 21.0 * y4 * y4
            env = jnp.where(y < 1.0, env, 0.0)
            o_v[row, pl.ds(col, LANES)] = 7.1998 * zfu * zfv * phi * env / dd
            return c
        lax.fori_loop(0, CH // LANES, _vec, 0)

        def _scat(j, c):
            pltpu.sync_copy(o_v.at[j], acc_sh.at[r_v.at[j]], add=True)
            return c
        lax.fori_loop(0, CH_ROWS, _scat, 0)
        return carry

    lax.fori_loop(0, CHUNKS, _chunk, 0)

    plsc.subcore_barrier()
    pltpu.sync_copy(acc_sh.at[pl.ds(sid * SEG, SEG)], zero_v)
    pltpu.sync_copy(zero_v, out_hbm.at[pl.ds(cid * ACC_N + sid * SEG, SEG)])


def kernel(z, edge_distance, edge_index, a_factor, Z_power, screen_coefs,
           screen_exps, covalent_radii):
    zpow_tab = jnp.arange(TAB, dtype=jnp.float32) ** Z_power
    rad_tab = jnp.pad(covalent_radii.astype(jnp.float32),
                      (0, TAB - covalent_radii.shape[0]))
    inv_ab = 1.0 / (0.529 * a_factor.astype(jnp.float32))
    par = jnp.concatenate([inv_ab[None], screen_coefs.astype(jnp.float32),
                           -screen_exps.astype(jnp.float32)])
    par2 = jnp.broadcast_to(par[:, None], (9, LANES)).astype(jnp.float32)

    pad = EP - N_EDGES
    send = jnp.concatenate(
        [edge_index[0], jnp.zeros((pad,), edge_index.dtype)]).reshape(ROWS, ROW)
    recv = jnp.concatenate(
        [edge_index[1], jnp.full((pad,), ACC_N - 1, edge_index.dtype)]
    ).reshape(ROWS, ROW)
    dist = jnp.concatenate(
        [edge_distance.astype(jnp.float32), jnp.ones((pad,), jnp.float32)]
    ).reshape(ROWS, ROW)

    mesh = plsc.VectorSubcoreMesh(core_axis_name="c", subcore_axis_name="s",
                                  num_cores=NC, num_subcores=NS)
    run = pl.kernel(
        _zbl_body,
        out_type=jax.ShapeDtypeStruct((NC * ACC_N,), jnp.float32),
        mesh=mesh,
        compiler_params=pltpu.CompilerParams(needs_layout_passes=False),
        scratch_types=[
            pltpu.VMEM((N_NODES,), jnp.int32),
            pltpu.VMEM((TAB,), jnp.float32),
            pltpu.VMEM((TAB,), jnp.float32),
            pltpu.VMEM((9, LANES), jnp.float32),
            pltpu.VMEM((CH_ROWS, ROW), jnp.int32),
            pltpu.VMEM((CH_ROWS, ROW), jnp.int32),
            pltpu.VMEM((CH_ROWS, ROW), jnp.float32),
            pltpu.VMEM((CH_ROWS, ROW), jnp.float32),
            pltpu.VMEM((SEG,), jnp.float32),
            pltpu.VMEM_SHARED((ACC_N,), jnp.float32),
        ],
    )
    partial = run(z.astype(jnp.int32), send, recv, dist, zpow_tab, rad_tab,
                  par2)
    return partial[:N_NODES] + partial[ACC_N:ACC_N + N_NODES]


# speedup vs baseline: 417.4735x; 1.4461x over previous
import jax
import jax.numpy as jnp
from jax import lax
from jax.experimental import pallas as pl
from jax.experimental.pallas import tpu as pltpu
from jax.experimental.pallas import tpu_sc as plsc

N_NODES = 100000
N_EDGES = 6400000
NC, NS, LANES = 2, 16, 16
NW = NC * NS
ROW = 128
CH_ROWS = 8
CH = CH_ROWS * ROW
CHUNKS = 198
TILE_E = CH * CHUNKS
EP = TILE_E * NW
ROWS = EP // ROW
TILE_ROWS = TILE_E // ROW
SEG = 6256
ACC_N = SEG * NS
TAB = 128
NBUF = 3


def _zbl_body(z_hbm, send_hbm, recv_hbm, dist_hbm, zpow_hbm, rad_hbm, par_hbm,
              out_hbm,
              z_v, zpow_v, rad_v, par_v, s_v, r_v, d_v, o_v, zero_v, acc_sh,
              in_sems, sc_sems):
    cid = lax.axis_index("c")
    sid = lax.axis_index("s")
    wid = sid * NC + cid
    row0 = wid * TILE_ROWS

    pltpu.sync_copy(z_hbm, z_v)
    pltpu.sync_copy(zpow_hbm, zpow_v)
    pltpu.sync_copy(rad_hbm, rad_v)
    pltpu.sync_copy(par_hbm, par_v)

    def issue_inputs(c, b):
        rb = row0 + c * CH_ROWS
        pltpu.async_copy(send_hbm.at[pl.ds(rb, CH_ROWS)], s_v.at[b],
                         in_sems.at[b])
        pltpu.async_copy(recv_hbm.at[pl.ds(rb, CH_ROWS)], r_v.at[b],
                         in_sems.at[b])
        pltpu.async_copy(dist_hbm.at[pl.ds(rb, CH_ROWS)], d_v.at[b],
                         in_sems.at[b])

    def wait_inputs(b):
        pltpu.make_async_copy(send_hbm.at[pl.ds(0, CH_ROWS)], s_v.at[b],
                              in_sems.at[b]).wait()
        pltpu.make_async_copy(recv_hbm.at[pl.ds(0, CH_ROWS)], r_v.at[b],
                              in_sems.at[b]).wait()
        pltpu.make_async_copy(dist_hbm.at[pl.ds(0, CH_ROWS)], d_v.at[b],
                              in_sems.at[b]).wait()

    def issue_scatter(b):
        for j in range(CH_ROWS):
            pltpu.async_copy(o_v.at[b, j], acc_sh.at[r_v.at[b, j]],
                             sc_sems.at[b], add=True)

    def wait_scatter(b):
        for j in range(CH_ROWS):
            pltpu.make_async_copy(o_v.at[b, j], acc_sh.at[r_v.at[b, j]],
                                  sc_sems.at[b]).wait()

    issue_inputs(0, 0)
    issue_inputs(1, 1)

    def _zero(i, c):
        zero_v[pl.ds(i * LANES, LANES)] = jnp.zeros((LANES,), jnp.float32)
        return c
    lax.fori_loop(0, SEG // LANES, _zero, 0)
    pltpu.sync_copy(zero_v, acc_sh.at[pl.ds(sid * SEG, SEG)])
    plsc.subcore_barrier()

    inv_ab = par_v[0]
    c0, c1, c2, c3 = par_v[1], par_v[2], par_v[3], par_v[4]
    nd0, nd1, nd2, nd3 = par_v[5], par_v[6], par_v[7], par_v[8]

    def compute_chunk(b):
        sref, rref, dref, oref = s_v.at[b], r_v.at[b], d_v.at[b], o_v.at[b]

        def _vec(i, c):
            row = i // (ROW // LANES)
            col = (i % (ROW // LANES)) * LANES
            s = sref[row, pl.ds(col, LANES)]
            r = rref[row, pl.ds(col, LANES)]
            dd = dref[row, pl.ds(col, LANES)]
            zu = plsc.load_gather(z_v, [s])
            zv = plsc.load_gather(z_v, [r])
            zfu = zu.astype(jnp.float32)
            zfv = zv.astype(jnp.float32)
            pu = plsc.load_gather(zpow_v, [zu])
            pv = plsc.load_gather(zpow_v, [zv])
            ru = plsc.load_gather(rad_v, [zu])
            rv = plsc.load_gather(rad_v, [zv])
            x = dd * ((pu + pv) * inv_ab)
            phi = (c0 * jnp.exp(nd0 * x) + c1 * jnp.exp(nd1 * x)
                   + c2 * jnp.exp(nd2 * x) + c3 * jnp.exp(nd3 * x))
            y = dd / (ru + rv)
            y2 = y * y
            y4 = y2 * y2
            y6 = y4 * y2
            env = 1.0 - 28.0 * y6 + 48.0 * y6 * y - 21.0 * y4 * y4
            env = jnp.where(y < 1.0, env, 0.0)
            oref[row, pl.ds(col, LANES)] = 7.1998 * zfu * zfv * phi * env / dd
            return c
        lax.fori_loop(0, CH // LANES, _vec, 0, unroll=2)

    def _group(g, carry):
        for b in range(NBUF):
            c = g * NBUF + b
            nb = (b + 2) % NBUF
            wait_inputs(b)
            compute_chunk(b)
            issue_scatter(b)
            @pl.when(c >= 1)
            def _():
                wait_scatter(nb)

            @pl.when(c + 2 < CHUNKS)
            def _():
                issue_inputs(c + 2, nb)
        return carry

    lax.fori_loop(0, CHUNKS // NBUF, _group, 0)
    wait_scatter((CHUNKS - 1) % NBUF)

    plsc.subcore_barrier()
    pltpu.sync_copy(acc_sh.at[pl.ds(sid * SEG, SEG)], zero_v)
    pltpu.sync_copy(zero_v, out_hbm.at[pl.ds(cid * ACC_N + sid * SEG, SEG)])


def kernel(z, edge_distance, edge_index, a_factor, Z_power, screen_coefs,
           screen_exps, covalent_radii):
    zpow_tab = jnp.arange(TAB, dtype=jnp.float32) ** Z_power
    rad_tab = jnp.pad(covalent_radii.astype(jnp.float32),
                      (0, TAB - covalent_radii.shape[0]))
    inv_ab = 1.0 / (0.529 * a_factor.astype(jnp.float32))
    par = jnp.concatenate([inv_ab[None], screen_coefs.astype(jnp.float32),
                           -screen_exps.astype(jnp.float32)])
    par2 = jnp.broadcast_to(par[:, None], (9, LANES)).astype(jnp.float32)

    pad = EP - N_EDGES
    send = jnp.concatenate(
        [edge_index[0], jnp.zeros((pad,), edge_index.dtype)]).reshape(ROWS, ROW)
    recv = jnp.concatenate(
        [edge_index[1], jnp.full((pad,), ACC_N - 1, edge_index.dtype)]
    ).reshape(ROWS, ROW)
    dist = jnp.concatenate(
        [edge_distance.astype(jnp.float32), jnp.ones((pad,), jnp.float32)]
    ).reshape(ROWS, ROW)

    mesh = plsc.VectorSubcoreMesh(core_axis_name="c", subcore_axis_name="s",
                                  num_cores=NC, num_subcores=NS)
    run = pl.kernel(
        _zbl_body,
        out_type=jax.ShapeDtypeStruct((NC * ACC_N,), jnp.float32),
        mesh=mesh,
        compiler_params=pltpu.CompilerParams(needs_layout_passes=False),
        scratch_types=[
            pltpu.VMEM((N_NODES,), jnp.int32),
            pltpu.VMEM((TAB,), jnp.float32),
            pltpu.VMEM((TAB,), jnp.float32),
            pltpu.VMEM((9, LANES), jnp.float32),
            pltpu.VMEM((NBUF, CH_ROWS, ROW), jnp.int32),
            pltpu.VMEM((NBUF, CH_ROWS, ROW), jnp.int32),
            pltpu.VMEM((NBUF, CH_ROWS, ROW), jnp.float32),
            pltpu.VMEM((NBUF, CH_ROWS, ROW), jnp.float32),
            pltpu.VMEM((SEG,), jnp.float32),
            pltpu.VMEM_SHARED((ACC_N,), jnp.float32),
            pltpu.SemaphoreType.DMA((NBUF,)),
            pltpu.SemaphoreType.DMA((NBUF,)),
        ],
    )
    partial = run(z.astype(jnp.int32), send, recv, dist, zpow_tab, rad_tab,
                  par2)
    return partial[:N_NODES] + partial[ACC_N:ACC_N + N_NODES]
